# baseline (speedup 1.0000x reference)
import functools

import jax
import jax.numpy as jnp
from jax import lax
from jax.experimental import pallas as pl
from jax.experimental.pallas import tpu as pltpu
from jax.experimental.pallas import tpu_sc as plsc

NC = 2
NS = 16
L = 16
NW = NC * NS
ROWS = 128
COLS = 32768
CPW = COLS // NW
CHUNK = 256
NCHUNK = CPW // CHUNK
NGROUP = CHUNK // L


def _ce(lst, i, j):
    a, b = lst[i], lst[j]
    lst[i] = jnp.minimum(a, b)
    lst[j] = jnp.maximum(a, b)


def _oddeven_merge(lst, lo, n, r):
    step = r * 2
    if step < n:
        _oddeven_merge(lst, lo, n, step)
        _oddeven_merge(lst, lo + r, n, step)
        for i in range(lo + r, lo + n - r, step):
            _ce(lst, i, i + r)
    else:
        _ce(lst, lo, lo + r)


def _oddeven_sort(lst, lo, n):
    if n > 1:
        m = n // 2
        _oddeven_sort(lst, lo, m)
        _oddeven_sort(lst, lo + m, m)
        _oddeven_merge(lst, lo, n, 1)


def _bitonic_clean(lst, lo, n):
    if n > 1:
        m = n // 2
        for i in range(lo, lo + m):
            _ce(lst, i, i + m)
        _bitonic_clean(lst, lo, m)
        _bitonic_clean(lst, lo + m, m)


def _top32_sorted(a, b):
    t = [jnp.maximum(a[i], b[31 - i]) for i in range(32)]
    _bitonic_clean(t, 0, 32)
    return t


def _group_threshold(load_row):
    def sorted16(i0):
        blk = [load_row(i0 + t) for t in range(16)]
        _oddeven_sort(blk, 0, 16)
        return blk

    def sorted32(i0):
        blk = sorted16(i0) + sorted16(i0 + 16)
        _oddeven_merge(blk, 0, 32, 1)
        return blk

    x32 = _top32_sorted(sorted32(0), sorted32(32))
    y32 = _top32_sorted(sorted32(64), sorted32(96))
    f = [jnp.maximum(x32[i], y32[31 - i]) for i in range(32)]
    while len(f) > 1:
        f = [jnp.minimum(f[2 * i], f[2 * i + 1]) for i in range(len(f) // 2)]
    return f[0]


def _make_kernel():
    mesh = plsc.VectorSubcoreMesh(
        core_axis_name="c", subcore_axis_name="s",
        num_cores=NC, num_subcores=NS)

    NBUF = 3

    @functools.partial(
        pl.kernel,
        out_type=jax.ShapeDtypeStruct((ROWS, COLS), jnp.float32),
        mesh=mesh,
        scratch_types=(
            [pltpu.VMEM((ROWS, CHUNK), jnp.float32)] * NBUF
            + [pltpu.SemaphoreType.DMA] * (2 * NBUF)),
        compiler_params=pltpu.CompilerParams(use_tc_tiling_on_sc=True),
    )
    def topk_mask(x_hbm, out_hbm, *scratch):
        bufs = scratch[:NBUF]
        sin = scratch[NBUF:2 * NBUF]
        sout = scratch[2 * NBUF:]
        wid = lax.axis_index("s") * NC + lax.axis_index("c")
        base_w = wid * CPW

        def copy_in(g):
            base = base_w + g * CHUNK
            return pltpu.async_copy(
                x_hbm.at[:, pl.ds(base, CHUNK)], bufs[g % NBUF],
                sin[g % NBUF])

        def copy_out(g):
            base = base_w + g * CHUNK
            return pltpu.async_copy(
                bufs[g % NBUF], out_hbm.at[:, pl.ds(base, CHUNK)],
                sout[g % NBUF])

        def compute(g):
            buf = bufs[g % NBUF]

            @plsc.parallel_loop(0, NGROUP, 1, unroll=2)
            def do_group(g2):
                c0 = pl.multiple_of(g2 * L, L)
                t = _group_threshold(lambda r: buf[r, pl.ds(c0, L)])
                for r in range(ROWS):
                    v = buf[r, pl.ds(c0, L)]
                    buf[r, pl.ds(c0, L)] = jnp.where(v >= t, v, 0.0)

        def do_chunk(g, carry):
            base = base_w + g * CHUNK
            pltpu.sync_copy(x_hbm.at[:, pl.ds(base, CHUNK)], bufs[0])
            compute(0)
            pltpu.sync_copy(bufs[0], out_hbm.at[:, pl.ds(base, CHUNK)])
            return carry

        lax.fori_loop(0, NCHUNK, do_chunk, 0)

    return topk_mask


_topk_mask = _make_kernel()


@jax.jit
def kernel(x):
    return _topk_mask(x)


# speedup vs baseline: 2.0111x; 2.0111x over previous
import functools

import jax
import jax.numpy as jnp
from jax import lax
from jax.experimental import pallas as pl
from jax.experimental.pallas import tpu as pltpu
from jax.experimental.pallas import tpu_sc as plsc

NC = 2
NS = 16
L = 16
NW = NC * NS
ROWS = 128
COLS = 32768
CPW = COLS // NW
CHUNK = 256
NCHUNK = CPW // CHUNK
NGROUP = CHUNK // L


def _ce(lst, i, j):
    a, b = lst[i], lst[j]
    lst[i] = jnp.minimum(a, b)
    lst[j] = jnp.maximum(a, b)


def _oddeven_merge(lst, lo, n, r):
    step = r * 2
    if step < n:
        _oddeven_merge(lst, lo, n, step)
        _oddeven_merge(lst, lo + r, n, step)
        for i in range(lo + r, lo + n - r, step):
            _ce(lst, i, i + r)
    else:
        _ce(lst, lo, lo + r)


def _oddeven_sort(lst, lo, n):
    if n > 1:
        m = n // 2
        _oddeven_sort(lst, lo, m)
        _oddeven_sort(lst, lo + m, m)
        _oddeven_merge(lst, lo, n, 1)


def _bitonic_clean(lst, lo, n):
    if n > 1:
        m = n // 2
        for i in range(lo, lo + m):
            _ce(lst, i, i + m)
        _bitonic_clean(lst, lo, m)
        _bitonic_clean(lst, lo + m, m)


def _top32_sorted(a, b):
    t = [jnp.maximum(a[i], b[31 - i]) for i in range(32)]
    _bitonic_clean(t, 0, 32)
    return t


def _group_threshold(load_row, stash_st, stash_ld):
    def sorted16(i0):
        blk = [load_row(i0 + t) for t in range(16)]
        _oddeven_sort(blk, 0, 16)
        return blk

    def sorted32(i0):
        blk = sorted16(i0) + sorted16(i0 + 16)
        _oddeven_merge(blk, 0, 32, 1)
        return blk

    x32 = _top32_sorted(sorted32(0), sorted32(32))
    for i in range(32):
        stash_st(i, x32[i])
    y32 = _top32_sorted(sorted32(64), sorted32(96))
    f = [jnp.maximum(stash_ld(i), y32[31 - i]) for i in range(32)]
    while len(f) > 1:
        f = [jnp.minimum(f[2 * i], f[2 * i + 1]) for i in range(len(f) // 2)]
    return f[0]


def _make_kernel():
    mesh = plsc.VectorSubcoreMesh(
        core_axis_name="c", subcore_axis_name="s",
        num_cores=NC, num_subcores=NS)

    NBUF = 3

    @functools.partial(
        pl.kernel,
        out_type=jax.ShapeDtypeStruct((ROWS, COLS), jnp.float32),
        mesh=mesh,
        scratch_types=(
            [pltpu.VMEM((ROWS, CHUNK), jnp.float32)] * NBUF
            + [pltpu.VMEM((32, CHUNK), jnp.float32)]
            + [pltpu.SemaphoreType.DMA] * (2 * NBUF)),
        compiler_params=pltpu.CompilerParams(use_tc_tiling_on_sc=True),
    )
    def topk_mask(x_hbm, out_hbm, *scratch):
        bufs = scratch[:NBUF]
        stash = scratch[NBUF]
        sin = scratch[NBUF + 1:NBUF + 1 + NBUF]
        sout = scratch[NBUF + 1 + NBUF:]
        wid = lax.axis_index("s") * NC + lax.axis_index("c")
        base_w = wid * CPW

        def copy_in(g):
            base = base_w + g * CHUNK
            return pltpu.async_copy(
                x_hbm.at[:, pl.ds(base, CHUNK)], bufs[g % NBUF],
                sin[g % NBUF])

        def copy_out(g):
            base = base_w + g * CHUNK
            return pltpu.async_copy(
                bufs[g % NBUF], out_hbm.at[:, pl.ds(base, CHUNK)],
                sout[g % NBUF])

        def compute(g):
            buf = bufs[g % NBUF]

            @plsc.parallel_loop(0, NGROUP, 1)
            def do_group(g2):
                c0 = pl.multiple_of(g2 * L, L)

                def stash_st(i, v):
                    stash[i, pl.ds(c0, L)] = v

                def stash_ld(i):
                    return stash[i, pl.ds(c0, L)]

                t = _group_threshold(
                    lambda r: buf[r, pl.ds(c0, L)], stash_st, stash_ld)
                for r in range(ROWS):
                    v = buf[r, pl.ds(c0, L)]
                    buf[r, pl.ds(c0, L)] = jnp.where(v >= t, v, 0.0)

        d_in = {g: copy_in(g) for g in range(min(NBUF, NCHUNK))}
        d_out = {}
        for g in range(NCHUNK):
            d_in[g].wait()
            compute(g)
            d_out[g] = copy_out(g)
            nxt = g + NBUF
            if nxt < NCHUNK:
                d_out[nxt - NBUF].wait()
                d_in[nxt] = copy_in(nxt)
        for g in range(max(0, NCHUNK - NBUF), NCHUNK):
            d_out[g].wait()

    return topk_mask


_topk_mask = _make_kernel()


@jax.jit
def kernel(x):
    return _topk_mask(x)


# speedup vs baseline: 2.3086x; 1.1479x over previous
import functools

import jax
import jax.numpy as jnp
from jax import lax
from jax.experimental import pallas as pl
from jax.experimental.pallas import tpu as pltpu
from jax.experimental.pallas import tpu_sc as plsc

NC = 2
NS = 16
L = 16
NW = NC * NS
ROWS = 128
COLS = 32768
CPW = COLS // NW
CHUNK = 256
NCHUNK = CPW // CHUNK
NGROUP = CHUNK // L


def _ce(lst, i, j):
    a, b = lst[i], lst[j]
    lst[i] = jnp.minimum(a, b)
    lst[j] = jnp.maximum(a, b)


def _oddeven_merge(lst, lo, n, r):
    step = r * 2
    if step < n:
        _oddeven_merge(lst, lo, n, step)
        _oddeven_merge(lst, lo + r, n, step)
        for i in range(lo + r, lo + n - r, step):
            _ce(lst, i, i + r)
    else:
        _ce(lst, lo, lo + r)


def _oddeven_sort(lst, lo, n):
    if n > 1:
        m = n // 2
        _oddeven_sort(lst, lo, m)
        _oddeven_sort(lst, lo + m, m)
        _oddeven_merge(lst, lo, n, 1)


def _bitonic_clean(lst, lo, n):
    if n > 1:
        m = n // 2
        for i in range(lo, lo + m):
            _ce(lst, i, i + m)
        _bitonic_clean(lst, lo, m)
        _bitonic_clean(lst, lo + m, m)


def _top32_sorted(a, b):
    t = [jnp.maximum(a[i], b[31 - i]) for i in range(32)]
    _bitonic_clean(t, 0, 32)
    return t


def _group_threshold(load_row):
    def sorted16(i0):
        blk = [load_row(i0 + t) for t in range(16)]
        _oddeven_sort(blk, 0, 16)
        return blk

    def sorted32(i0):
        blk = sorted16(i0) + sorted16(i0 + 16)
        _oddeven_merge(blk, 0, 32, 1)
        return blk

    x32 = _top32_sorted(sorted32(0), sorted32(32))
    y32 = _top32_sorted(sorted32(64), sorted32(96))
    f = [jnp.maximum(x32[i], y32[31 - i]) for i in range(32)]
    while len(f) > 1:
        f = [jnp.minimum(f[2 * i], f[2 * i + 1]) for i in range(len(f) // 2)]
    return f[0]


def _make_kernel():
    mesh = plsc.VectorSubcoreMesh(
        core_axis_name="c", subcore_axis_name="s",
        num_cores=NC, num_subcores=NS)

    NBUF = 3

    @functools.partial(
        pl.kernel,
        out_type=jax.ShapeDtypeStruct((ROWS, COLS), jnp.float32),
        mesh=mesh,
        scratch_types=(
            [pltpu.VMEM((ROWS, CHUNK), jnp.float32)] * NBUF
            + [pltpu.SemaphoreType.DMA] * (2 * NBUF)),
        compiler_params=pltpu.CompilerParams(use_tc_tiling_on_sc=True),
    )
    def topk_mask(x_hbm, out_hbm, *scratch):
        bufs = scratch[:NBUF]
        sin = scratch[NBUF:2 * NBUF]
        sout = scratch[2 * NBUF:]
        wid = lax.axis_index("s") * NC + lax.axis_index("c")
        base_w = wid * CPW

        def copy_in(g):
            base = base_w + g * CHUNK
            return pltpu.async_copy(
                x_hbm.at[:, pl.ds(base, CHUNK)], bufs[g % NBUF],
                sin[g % NBUF])

        def copy_out(g):
            base = base_w + g * CHUNK
            return pltpu.async_copy(
                bufs[g % NBUF], out_hbm.at[:, pl.ds(base, CHUNK)],
                sout[g % NBUF])

        def compute(g):
            buf = bufs[g % NBUF]

            @plsc.parallel_loop(0, NGROUP, 1)
            def do_group(g2):
                c0 = pl.multiple_of(g2 * L, L)

                t = _group_threshold(lambda r: buf[r, pl.ds(c0, L)])
                for r in range(ROWS):
                    v = buf[r, pl.ds(c0, L)]
                    buf[r, pl.ds(c0, L)] = jnp.where(v >= t, v, 0.0)

        d_in = {g: copy_in(g) for g in range(min(NBUF, NCHUNK))}
        d_out = {}
        for g in range(NCHUNK):
            d_in[g].wait()
            compute(g)
            d_out[g] = copy_out(g)
            nxt = g + NBUF
            if nxt < NCHUNK:
                d_out[nxt - NBUF].wait()
                d_in[nxt] = copy_in(nxt)
        for g in range(max(0, NCHUNK - NBUF), NCHUNK):
            d_out[g].wait()

    return topk_mask


_topk_mask = _make_kernel()


@jax.jit
def kernel(x):
    return _topk_mask(x)
